# baseline (speedup 1.0000x reference)
import functools

import jax
import jax.numpy as jnp
from jax import lax
from jax.experimental import pallas as pl
from jax.experimental.pallas import tpu as pltpu
from jax.experimental.pallas import tpu_sc as plsc

N = 50000
M = 16
WL = 8
PW = 2 * WL
NB = 16
NBLOCKS = N // NB
NC, NS, L = 2, 16, 16
NW = NC * NS
NG = NB // L


def _lane_reduce(v, op):
    iota = lax.iota(jnp.int32, L)
    for d in (8, 4, 2, 1):
        v = op(v, v.at[iota ^ d].get(mode="promise_in_bounds",
                                     unique_indices=True))
    return v


def _f_groups(xb, a_v, m_vec, n_vec, zero_v):
    acc = None
    for w in range(WL):
        w_vec = zero_v + w
        i1 = plsc.load_gather(xb, [m_vec, n_vec, w_vec, zero_v])
        i2 = plsc.load_gather(xb, [m_vec, n_vec, w_vec, zero_v + 1])
        y1 = plsc.load_gather(a_v, [i1])
        y2 = plsc.load_gather(a_v, [i2])
        z = y1 * y2
        acc = z if acc is None else jnp.maximum(acc, z)
    return acc


def _compute_f(xb, a_v, f_v, iota, zero_v):
    def mbody(m, carry):
        m_vec = jnp.broadcast_to(m, (L,)).astype(jnp.int32)
        for g in range(NG):
            n_vec = iota + g * L
            f_v[m, pl.ds(g * L, L)] = _f_groups(xb, a_v, m_vec, n_vec,
                                                zero_v)
        return carry
    lax.fori_loop(0, M, mbody, 0, unroll=False)


def _tec_body(a_hbm, x1_hbm, x2_hbm, w_hbm, out_hbm,
              a_v, xb, f1_v, f2_v, pi_v, out_v):
    cid = lax.axis_index("c")
    sid = lax.axis_index("s")
    wid = sid * NC + cid

    pltpu.sync_copy(a_hbm, a_v)
    pltpu.sync_copy(w_hbm, pi_v)

    rows = [pi_v[i, :] for i in range(M)]
    mx = _lane_reduce(functools.reduce(jnp.maximum, rows), jnp.maximum)
    mxs = mx[0]
    es = [jnp.exp(r - mxs) for r in rows]
    tot = functools.reduce(lambda x, y: x + y, es)
    inv_v = 1.0 / _lane_reduce(tot, lambda x, y: x + y)
    pis = [e * inv_v for e in es]
    pi1 = [_lane_reduce(p, lambda x, y: x + y)[0] for p in pis]
    pi2v = tot * inv_v
    pi2 = [pi2v[m2] for m2 in range(M)]
    pi_s = [[pis[m1][m2] for m2 in range(M)] for m1 in range(M)]

    iota = lax.iota(jnp.int32, L)
    zero_v = jnp.zeros((L,), jnp.int32)

    num_j = (NBLOCKS - wid + NW - 1) // NW

    def blk(j, carry):
        b = wid + j * NW
        n0 = b * NB

        pltpu.sync_copy(x1_hbm.at[:, pl.ds(n0, NB), :, :], xb)
        _compute_f(xb, a_v, f1_v, iota, zero_v)
        pltpu.sync_copy(x2_hbm.at[:, pl.ds(n0, NB), :, :], xb)
        _compute_f(xb, a_v, f2_v, iota, zero_v)

        for g in range(NG):
            f1g = [f1_v[m, pl.ds(g * L, L)] for m in range(M)]
            f2g = [f2_v[m, pl.ds(g * L, L)] for m in range(M)]
            eu = functools.reduce(
                lambda x, y: x + y, [pi1[m] * f1g[m] for m in range(M)])
            ev = functools.reduce(
                lambda x, y: x + y, [pi2[m] * f2g[m] for m in range(M)])
            euv = None
            for m1 in range(M):
                mrow = None
                for m2 in range(M):
                    t = pi_s[m1][m2] * f2g[m2]
                    mrow = t if mrow is None else mrow + t
                t = f1g[m1] * mrow
                euv = t if euv is None else euv + t
            fp = eu + ev - euv
            av = a_v[pl.ds(n0 + g * L, L)]
            out_v[pl.ds(g * L, L)] = 1.0 - (1.0 - av) * (1.0 - fp)

        pltpu.sync_copy(out_v, out_hbm.at[pl.ds(n0, NB)])
        return carry

    lax.fori_loop(0, num_j, blk, 0)


@jax.jit
def _run(a, x1r, x2r, w):
    mesh = plsc.VectorSubcoreMesh(
        core_axis_name="c", subcore_axis_name="s",
        num_cores=NC, num_subcores=NS)
    return pl.kernel(
        _tec_body,
        out_type=jax.ShapeDtypeStruct((N,), jnp.float32),
        mesh=mesh,
        compiler_params=pltpu.CompilerParams(use_tc_tiling_on_sc=False, needs_layout_passes=False),
        scratch_types=[
            pltpu.VMEM((N,), jnp.float32),
            pltpu.VMEM((M, NB, WL, 2), jnp.int32),
            pltpu.VMEM((M, NB), jnp.float32),
            pltpu.VMEM((M, NB), jnp.float32),
            pltpu.VMEM((M, M), jnp.float32),
            pltpu.VMEM((NB,), jnp.float32),
        ],
    )(a, x1r, x2r, w)


def kernel(a, X1, X2, W):
    return _run(a, X1, X2, W)


# speedup vs baseline: 8.2312x; 8.2312x over previous
import functools

import jax
import jax.numpy as jnp
from jax import lax
from jax.experimental import pallas as pl
from jax.experimental.pallas import tpu as pltpu
from jax.experimental.pallas import tpu_sc as plsc

N = 50000
M = 16
WL = 8
PW = 2 * WL
NB = 80
NBLOCKS = N // NB
NC, NS, L = 2, 16, 16
NW = NC * NS
NG = NB // L


def _lane_reduce(v, op):
    iota = lax.iota(jnp.int32, L)
    for d in (8, 4, 2, 1):
        v = op(v, v.at[iota ^ d].get(mode="promise_in_bounds",
                                     unique_indices=True))
    return v


def _f_group(xb, a_v, m_vec, col_g):
    idxs = [plsc.load_gather(xb, [m_vec, col_g + k]) for k in range(PW)]
    ys = [plsc.load_gather(a_v, [i]) for i in idxs]
    zs = [ys[2 * t] * ys[2 * t + 1] for t in range(WL)]
    while len(zs) > 1:
        zs = [jnp.maximum(zs[2 * i], zs[2 * i + 1])
              for i in range(len(zs) // 2)]
    return zs[0]


def _compute_f(xb, a_v, f_v, colbase):
    def mbody(m, carry):
        m_vec = jnp.broadcast_to(m, (L,)).astype(jnp.int32)
        for g in range(NG):
            f_v[m, pl.ds(g * L, L)] = _f_group(xb, a_v, m_vec,
                                               colbase + g * (L * PW))
        return carry
    lax.fori_loop(0, M, mbody, 0, unroll=2)


def _tec_body(a_hbm, x1_hbm, x2_hbm, w_hbm, out_hbm,
              a_v, xb, f1_v, f2_v, pi_v, out_v):
    cid = lax.axis_index("c")
    sid = lax.axis_index("s")
    wid = sid * NC + cid

    pltpu.sync_copy(a_hbm, a_v)
    pltpu.sync_copy(w_hbm, pi_v)

    rows = [pi_v[i, :] for i in range(M)]
    mx = _lane_reduce(functools.reduce(jnp.maximum, rows), jnp.maximum)
    mxs = mx[0]
    es = [jnp.exp(r - mxs) for r in rows]
    tot = functools.reduce(lambda x, y: x + y, es)
    inv_v = 1.0 / _lane_reduce(tot, lambda x, y: x + y)
    pis = [e * inv_v for e in es]
    pi1 = [_lane_reduce(p, lambda x, y: x + y)[0] for p in pis]
    pi2v = tot * inv_v
    pi2 = [pi2v[m2] for m2 in range(M)]
    pi_s = [[pis[m1][m2] for m2 in range(M)] for m1 in range(M)]

    iota = lax.iota(jnp.int32, L)
    colbase = iota * PW

    num_j = (NBLOCKS - wid + NW - 1) // NW

    def blk(j, carry):
        b = wid + j * NW
        n0 = b * NB

        pltpu.sync_copy(x1_hbm.at[:, pl.ds(n0 * PW, NB * PW)], xb)
        _compute_f(xb, a_v, f1_v, colbase)
        pltpu.sync_copy(x2_hbm.at[:, pl.ds(n0 * PW, NB * PW)], xb)
        _compute_f(xb, a_v, f2_v, colbase)

        for g in range(NG):
            f1g = [f1_v[m, pl.ds(g * L, L)] for m in range(M)]
            f2g = [f2_v[m, pl.ds(g * L, L)] for m in range(M)]
            eu = functools.reduce(
                lambda x, y: x + y, [pi1[m] * f1g[m] for m in range(M)])
            ev = functools.reduce(
                lambda x, y: x + y, [pi2[m] * f2g[m] for m in range(M)])
            euv = None
            for m1 in range(M):
                mrow = None
                for m2 in range(M):
                    t = pi_s[m1][m2] * f2g[m2]
                    mrow = t if mrow is None else mrow + t
                t = f1g[m1] * mrow
                euv = t if euv is None else euv + t
            fp = eu + ev - euv
            av = a_v[pl.ds(n0 + g * L, L)]
            out_v[pl.ds(g * L, L)] = 1.0 - (1.0 - av) * (1.0 - fp)

        pltpu.sync_copy(out_v, out_hbm.at[pl.ds(n0, NB)])
        return carry

    lax.fori_loop(0, num_j, blk, 0)


@jax.jit
def _run(a, x1r, x2r, w):
    mesh = plsc.VectorSubcoreMesh(
        core_axis_name="c", subcore_axis_name="s",
        num_cores=NC, num_subcores=NS)
    return pl.kernel(
        _tec_body,
        out_type=jax.ShapeDtypeStruct((N,), jnp.float32),
        mesh=mesh,
        compiler_params=pltpu.CompilerParams(
            use_tc_tiling_on_sc=False, needs_layout_passes=False),
        scratch_types=[
            pltpu.VMEM((N,), jnp.float32),
            pltpu.VMEM((M, NB * PW), jnp.int32),
            pltpu.VMEM((M, NB), jnp.float32),
            pltpu.VMEM((M, NB), jnp.float32),
            pltpu.VMEM((M, M), jnp.float32),
            pltpu.VMEM((NB,), jnp.float32),
        ],
    )(a, x1r, x2r, w)


def kernel(a, X1, X2, W):
    x1r = X1.reshape(M, N * PW)
    x2r = X2.reshape(M, N * PW)
    return _run(a, x1r, x2r, W)


# speedup vs baseline: 17.2791x; 2.0992x over previous
import functools

import jax
import jax.numpy as jnp
from jax import lax
from jax.experimental import pallas as pl
from jax.experimental.pallas import tpu as pltpu
from jax.experimental.pallas import tpu_sc as plsc

N = 50000
M = 16
WL = 8
PW = 2 * WL
NB = 80
NBLOCKS = N // NB
NC, NS, L = 2, 16, 16
NW = NC * NS
NG = NB // L


def _lane_reduce(v, op):
    iota = lax.iota(jnp.int32, L)
    for d in (8, 4, 2, 1):
        v = op(v, v.at[iota ^ d].get(mode="promise_in_bounds",
                                     unique_indices=True))
    return v


def _f_group(xb, a_v, col_g):
    idxs = [plsc.load_gather(xb, [col_g + k]) for k in range(PW)]
    ys = [plsc.load_gather(a_v, [i]) for i in idxs]
    zs = [ys[2 * t] * ys[2 * t + 1] for t in range(WL)]
    while len(zs) > 1:
        zs = [jnp.maximum(zs[2 * i], zs[2 * i + 1])
              for i in range(len(zs) // 2)]
    return zs[0]


def _compute_f(xb, a_v, f_v, colbase):
    def mbody(m, carry):
        row0 = m * (NB * PW)
        for g in range(NG):
            f_v[pl.ds(m * NB + g * L, L)] = _f_group(
                xb, a_v, colbase + (row0 + g * (L * PW)))
        return carry
    lax.fori_loop(0, M, mbody, 0, unroll=2)


def _tec_body(a_hbm, x1_hbm, x2_hbm, w_hbm, out_hbm,
              a_v, xb, f1_v, f2_v, pi_v, out_v):
    cid = lax.axis_index("c")
    sid = lax.axis_index("s")
    wid = sid * NC + cid

    pltpu.sync_copy(a_hbm, a_v)
    pltpu.sync_copy(w_hbm, pi_v)

    rows = [pi_v[pl.ds(16 * i, L)] for i in range(M)]
    mx = _lane_reduce(functools.reduce(jnp.maximum, rows), jnp.maximum)
    mxs = mx[0]
    es = [jnp.exp(r - mxs) for r in rows]
    tot = functools.reduce(lambda x, y: x + y, es)
    inv_v = 1.0 / _lane_reduce(tot, lambda x, y: x + y)
    pis = [e * inv_v for e in es]
    pi1 = [_lane_reduce(p, lambda x, y: x + y)[0] for p in pis]
    pi2v = tot * inv_v
    pi2 = [pi2v[m2] for m2 in range(M)]
    pi_s = [[pis[m1][m2] for m2 in range(M)] for m1 in range(M)]

    iota = lax.iota(jnp.int32, L)
    colbase = iota * PW

    num_j = (NBLOCKS - wid + NW - 1) // NW

    def load_x(x_hbm, n0):
        for m in range(M):
            pltpu.sync_copy(x_hbm.at[m, pl.ds(n0 * PW, NB * PW)],
                            xb.at[pl.ds(m * (NB * PW), NB * PW)])

    def blk(j, carry):
        b = wid + j * NW
        n0 = b * NB

        load_x(x1_hbm, n0)
        _compute_f(xb, a_v, f1_v, colbase)
        load_x(x2_hbm, n0)
        _compute_f(xb, a_v, f2_v, colbase)

        for g in range(NG):
            f1g = [f1_v[pl.ds(m * NB + g * L, L)] for m in range(M)]
            f2g = [f2_v[pl.ds(m * NB + g * L, L)] for m in range(M)]
            eu = functools.reduce(
                lambda x, y: x + y, [pi1[m] * f1g[m] for m in range(M)])
            ev = functools.reduce(
                lambda x, y: x + y, [pi2[m] * f2g[m] for m in range(M)])
            euv = None
            for m1 in range(M):
                mrow = None
                for m2 in range(M):
                    t = pi_s[m1][m2] * f2g[m2]
                    mrow = t if mrow is None else mrow + t
                t = f1g[m1] * mrow
                euv = t if euv is None else euv + t
            fp = eu + ev - euv
            av = a_v[pl.ds(n0 + g * L, L)]
            out_v[pl.ds(g * L, L)] = 1.0 - (1.0 - av) * (1.0 - fp)

        pltpu.sync_copy(out_v, out_hbm.at[pl.ds(n0, NB)])
        return carry

    lax.fori_loop(0, num_j, blk, 0)


@jax.jit
def _run(a, x1r, x2r, w):
    mesh = plsc.VectorSubcoreMesh(
        core_axis_name="c", subcore_axis_name="s",
        num_cores=NC, num_subcores=NS)
    return pl.kernel(
        _tec_body,
        out_type=jax.ShapeDtypeStruct((N,), jnp.float32),
        mesh=mesh,
        compiler_params=pltpu.CompilerParams(
            use_tc_tiling_on_sc=True, needs_layout_passes=False),
        scratch_types=[
            pltpu.VMEM((N,), jnp.float32),
            pltpu.VMEM((M * NB * PW,), jnp.int32),
            pltpu.VMEM((M * NB,), jnp.float32),
            pltpu.VMEM((M * NB,), jnp.float32),
            pltpu.VMEM((M * M,), jnp.float32),
            pltpu.VMEM((NB,), jnp.float32),
        ],
    )(a, x1r, x2r, w)


def kernel(a, X1, X2, W):
    x1r = X1.reshape(M, N * PW)
    x2r = X2.reshape(M, N * PW)
    return _run(a, x1r, x2r, W.reshape(M * M))


# speedup vs baseline: 25.7639x; 1.4910x over previous
import functools

import jax
import jax.numpy as jnp
from jax import lax
from jax.experimental import pallas as pl
from jax.experimental.pallas import tpu as pltpu
from jax.experimental.pallas import tpu_sc as plsc

N = 50000
M = 16
WL = 8
PW = 2 * WL
NB = 80
NBLOCKS = N // NB
NC, NS, L = 2, 16, 16
NW = NC * NS
NG = NB // L


def _lane_reduce(v, op):
    iota = lax.iota(jnp.int32, L)
    for d in (8, 4, 2, 1):
        v = op(v, v.at[iota ^ d].get(mode="promise_in_bounds",
                                     unique_indices=True))
    return v


def _f_group(xb, a_v, col_g):
    idxs = [plsc.load_gather(xb, [col_g + k]) for k in range(PW)]
    ys = [plsc.load_gather(a_v, [i]) for i in idxs]
    zs = [ys[2 * t] * ys[2 * t + 1] for t in range(WL)]
    while len(zs) > 1:
        zs = [jnp.maximum(zs[2 * i], zs[2 * i + 1])
              for i in range(len(zs) // 2)]
    return zs[0]


def _compute_f(xb, a_v, f_v, colbase):
    def mbody(m, carry):
        row0 = m * (NB * PW)
        for g in range(NG):
            f_v[pl.ds(m * NB + g * L, L)] = _f_group(
                xb, a_v, colbase + (row0 + g * (L * PW)))
        return carry
    lax.fori_loop(0, M, mbody, 0, unroll=2)


def _tec_body(a_hbm, x1_hbm, x2_hbm, w_hbm, out_hbm,
              a_v, xb, xb2, f1_v, f2_v, pi_v, out_v, sem1, sem2):
    cid = lax.axis_index("c")
    sid = lax.axis_index("s")
    wid = sid * NC + cid

    pltpu.sync_copy(a_hbm, a_v)
    pltpu.sync_copy(w_hbm, pi_v)

    rows = [pi_v[pl.ds(16 * i, L)] for i in range(M)]
    mx = _lane_reduce(functools.reduce(jnp.maximum, rows), jnp.maximum)
    mxs = mx[0]
    es = [jnp.exp(r - mxs) for r in rows]
    tot = functools.reduce(lambda x, y: x + y, es)
    inv_v = 1.0 / _lane_reduce(tot, lambda x, y: x + y)
    pis = [e * inv_v for e in es]
    pi1 = [_lane_reduce(p, lambda x, y: x + y)[0] for p in pis]
    pi2v = tot * inv_v
    pi2 = [pi2v[m2] for m2 in range(M)]
    pi_s = [[pis[m1][m2] for m2 in range(M)] for m1 in range(M)]

    iota = lax.iota(jnp.int32, L)
    colbase = iota * PW

    num_j = (NBLOCKS - wid + NW - 1) // NW

    def x_dmas(x_hbm, n0, buf, sem):
        return [pltpu.make_async_copy(
                    x_hbm.at[m, pl.ds(n0 * PW, NB * PW)],
                    buf.at[pl.ds(m * (NB * PW), NB * PW)], sem)
                for m in range(M)]

    def start_x(x_hbm, n0, buf, sem):
        for c in x_dmas(x_hbm, n0, buf, sem):
            c.start()

    def wait_x(x_hbm, n0, buf, sem):
        for c in x_dmas(x_hbm, n0, buf, sem):
            c.wait()

    start_x(x1_hbm, wid * NB, xb, sem1)

    def blk(j, carry):
        b = wid + j * NW
        n0 = b * NB

        start_x(x2_hbm, n0, xb2, sem2)
        wait_x(x1_hbm, n0, xb, sem1)
        _compute_f(xb, a_v, f1_v, colbase)

        @pl.when(j + 1 < num_j)
        def _():
            start_x(x1_hbm, n0 + NW * NB, xb, sem1)

        wait_x(x2_hbm, n0, xb2, sem2)
        _compute_f(xb2, a_v, f2_v, colbase)

        for g in range(NG):
            f1g = [f1_v[pl.ds(m * NB + g * L, L)] for m in range(M)]
            f2g = [f2_v[pl.ds(m * NB + g * L, L)] for m in range(M)]
            eu = functools.reduce(
                lambda x, y: x + y, [pi1[m] * f1g[m] for m in range(M)])
            ev = functools.reduce(
                lambda x, y: x + y, [pi2[m] * f2g[m] for m in range(M)])
            euv = None
            for m1 in range(M):
                mrow = None
                for m2 in range(M):
                    t = pi_s[m1][m2] * f2g[m2]
                    mrow = t if mrow is None else mrow + t
                t = f1g[m1] * mrow
                euv = t if euv is None else euv + t
            fp = eu + ev - euv
            av = a_v[pl.ds(n0 + g * L, L)]
            out_v[pl.ds(g * L, L)] = 1.0 - (1.0 - av) * (1.0 - fp)

        pltpu.sync_copy(out_v, out_hbm.at[pl.ds(n0, NB)])
        return carry

    lax.fori_loop(0, num_j, blk, 0)


@jax.jit
def _run(a, x1r, x2r, w):
    mesh = plsc.VectorSubcoreMesh(
        core_axis_name="c", subcore_axis_name="s",
        num_cores=NC, num_subcores=NS)
    return pl.kernel(
        _tec_body,
        out_type=jax.ShapeDtypeStruct((N,), jnp.float32),
        mesh=mesh,
        compiler_params=pltpu.CompilerParams(
            use_tc_tiling_on_sc=True, needs_layout_passes=False),
        scratch_types=[
            pltpu.VMEM((N,), jnp.float32),
            pltpu.VMEM((M * NB * PW,), jnp.int32),
            pltpu.VMEM((M * NB * PW,), jnp.int32),
            pltpu.VMEM((M * NB,), jnp.float32),
            pltpu.VMEM((M * NB,), jnp.float32),
            pltpu.VMEM((M * M,), jnp.float32),
            pltpu.VMEM((NB,), jnp.float32),
            pltpu.SemaphoreType.DMA,
            pltpu.SemaphoreType.DMA,
        ],
    )(a, x1r, x2r, w)


def kernel(a, X1, X2, W):
    x1r = X1.reshape(M, N * PW)
    x2r = X2.reshape(M, N * PW)
    return _run(a, x1r, x2r, W.reshape(M * M))


# speedup vs baseline: 45.6135x; 1.7704x over previous
import functools

import jax
import jax.numpy as jnp
from jax import lax
from jax.experimental import pallas as pl
from jax.experimental.pallas import tpu as pltpu
from jax.experimental.pallas import tpu_sc as plsc

N = 50000
M = 16
WL = 8
PW = 2 * WL
NB = 128
NMAIN = N // NB
NT = N - NMAIN * NB
NC, NS, L = 2, 16, 16
NW = NC * NS
TAIL_W = NW - 1


def _lane_reduce(v, op):
    iota = lax.iota(jnp.int32, L)
    for d in (8, 4, 2, 1):
        v = op(v, v.at[iota ^ d].get(mode="promise_in_bounds",
                                     unique_indices=True))
    return v


def _f_group(xb, a_v, m, g):
    idxs = [xb[m, k, pl.ds(g * L, L)] for k in range(PW)]
    ys = [plsc.load_gather(a_v, [i]) for i in idxs]
    zs = [ys[2 * t] * ys[2 * t + 1] for t in range(WL)]
    while len(zs) > 1:
        zs = [jnp.maximum(zs[2 * i], zs[2 * i + 1])
              for i in range(len(zs) // 2)]
    return zs[0]


def _compute_f(xb, a_v, f_v, ng):
    def mbody(m, carry):
        for g in range(ng):
            f_v[pl.ds(m * NB + g * L, L)] = _f_group(xb, a_v, m, g)
        return carry
    lax.fori_loop(0, M, mbody, 0, unroll=2)


def _tec_body(a_hbm, x1_hbm, x2_hbm, x1t_hbm, x2t_hbm, w_hbm, out_hbm,
              a_v, xb, xb2, f1_v, f2_v, pi_v, out_v,
              sem1, sem2):
    cid = lax.axis_index("c")
    sid = lax.axis_index("s")
    wid = sid * NC + cid

    pltpu.sync_copy(a_hbm, a_v)
    pltpu.sync_copy(w_hbm, pi_v)

    rows = [pi_v[pl.ds(16 * i, L)] for i in range(M)]
    mx = _lane_reduce(functools.reduce(jnp.maximum, rows), jnp.maximum)
    mxs = mx[0]
    es = [jnp.exp(r - mxs) for r in rows]
    tot = functools.reduce(lambda x, y: x + y, es)
    inv_v = 1.0 / _lane_reduce(tot, lambda x, y: x + y)
    pis = [e * inv_v for e in es]
    pi1 = [_lane_reduce(p, lambda x, y: x + y)[0] for p in pis]
    pi2v = tot * inv_v
    pi2 = [pi2v[m2] for m2 in range(M)]
    pi_s = [[pis[m1][m2] for m2 in range(M)] for m1 in range(M)]

    num_j = (NMAIN - wid + NW - 1) // NW

    def x_dma(x_hbm, n0, nb, buf, sem):
        return pltpu.make_async_copy(
            x_hbm.at[:, :, pl.ds(n0, nb)],
            buf.at[:, :, pl.ds(0, nb)], sem)

    def combine_group(n0, g, f1_v_, f2_v_):
        f1g = [f1_v_[pl.ds(m * NB + g * L, L)] for m in range(M)]
        f2g = [f2_v_[pl.ds(m * NB + g * L, L)] for m in range(M)]
        eu = functools.reduce(
            lambda x, y: x + y, [pi1[m] * f1g[m] for m in range(M)])
        ev = functools.reduce(
            lambda x, y: x + y, [pi2[m] * f2g[m] for m in range(M)])
        euv = None
        for m1 in range(M):
            mrow = None
            for m2 in range(M):
                t = pi_s[m1][m2] * f2g[m2]
                mrow = t if mrow is None else mrow + t
            t = f1g[m1] * mrow
            euv = t if euv is None else euv + t
        fp = eu + ev - euv
        av = a_v[pl.ds(n0 + g * L, L)]
        out_v[pl.ds(g * L, L)] = 1.0 - (1.0 - av) * (1.0 - fp)

    x_dma(x1_hbm, pl.multiple_of(wid * NB, NB), NB, xb, sem1).start()

    def blk(j, carry):
        b = wid + j * NW
        n0 = pl.multiple_of(b * NB, NB)

        x_dma(x2_hbm, n0, NB, xb2, sem2).start()
        x_dma(x1_hbm, n0, NB, xb, sem1).wait()
        _compute_f(xb, a_v, f1_v, NB // L)

        @pl.when(j + 1 < num_j)
        def _():
            x_dma(x1_hbm, pl.multiple_of(n0 + NW * NB, NB), NB,
                  xb, sem1).start()

        x_dma(x2_hbm, n0, NB, xb2, sem2).wait()
        _compute_f(xb2, a_v, f2_v, NB // L)

        for g in range(NB // L):
            combine_group(n0, g, f1_v, f2_v)
        pltpu.sync_copy(out_v.at[pl.ds(0, NB)], out_hbm.at[pl.ds(n0, NB)])
        return carry

    lax.fori_loop(0, num_j, blk, 0)

    @pl.when(wid == TAIL_W)
    def _tail():
        n0 = NMAIN * NB
        pltpu.sync_copy(x1t_hbm, xb)
        pltpu.sync_copy(x2t_hbm, xb2)
        _compute_f(xb, a_v, f1_v, NT // L)
        _compute_f(xb2, a_v, f2_v, NT // L)
        for g in range(NT // L):
            combine_group(n0, g, f1_v, f2_v)
        pltpu.sync_copy(out_v.at[pl.ds(0, NT)], out_hbm.at[pl.ds(n0, NT)])


@jax.jit
def _run(a, x1t, x2t, x1tail, x2tail, w):
    mesh = plsc.VectorSubcoreMesh(
        core_axis_name="c", subcore_axis_name="s",
        num_cores=NC, num_subcores=NS)
    return pl.kernel(
        _tec_body,
        out_type=jax.ShapeDtypeStruct((N,), jnp.float32),
        mesh=mesh,
        compiler_params=pltpu.CompilerParams(
            use_tc_tiling_on_sc=True, needs_layout_passes=False),
        scratch_types=[
            pltpu.VMEM((N,), jnp.float32),
            pltpu.VMEM((M, PW, NB), jnp.int32),
            pltpu.VMEM((M, PW, NB), jnp.int32),
            pltpu.VMEM((M * NB,), jnp.float32),
            pltpu.VMEM((M * NB,), jnp.float32),
            pltpu.VMEM((M * M,), jnp.float32),
            pltpu.VMEM((NB,), jnp.float32),
            pltpu.SemaphoreType.DMA,
            pltpu.SemaphoreType.DMA,
        ],
    )(a, x1t, x2t, x1tail, x2tail, w)


def kernel(a, X1, X2, W):
    x1t = X1.transpose(0, 2, 3, 1).reshape(M, PW, N)
    x2t = X2.transpose(0, 2, 3, 1).reshape(M, PW, N)
    x1tail = jnp.pad(x1t[:, :, NMAIN * NB:], ((0, 0), (0, 0), (0, NB - NT)))
    x2tail = jnp.pad(x2t[:, :, NMAIN * NB:], ((0, 0), (0, 0), (0, NB - NT)))
    return _run(a, x1t, x2t, x1tail, x2tail, W.reshape(M * M))
